# baseline (speedup 1.0000x reference)
import functools

import jax
import jax.numpy as jnp
from jax import lax
from jax.experimental import pallas as pl
from jax.experimental.pallas import tpu as pltpu
from jax.experimental.pallas import tpu_sc as plsc

N = 10000
NP = 10240
E = 320000
D = 128
NC = 2
NS = 16
NW = NC * NS
EPT = E // NW
CHUNK = 80
NCHUNK = EPT // CHUNK
STRIPE = 624
L = 16


BLK = 2000


def _pre_body(x_ref, w_ref, av_ref, h_ref, as_ref, ad_ref):
    h = jnp.dot(x_ref[...], w_ref[...], preferred_element_type=jnp.float32)
    h_ref[...] = h
    as_ref[...] = jnp.sum(h * av_ref[0:1, :], axis=1, keepdims=True)
    ad_ref[...] = jnp.sum(h * av_ref[1:2, :], axis=1, keepdims=True)


def _pre_call(x, W, av):
    return pl.pallas_call(
        _pre_body,
        grid=(N // BLK,),
        in_specs=[
            pl.BlockSpec((BLK, D), lambda i: (i, 0)),
            pl.BlockSpec((D, D), lambda i: (0, 0)),
            pl.BlockSpec((2, D), lambda i: (0, 0)),
        ],
        out_specs=[
            pl.BlockSpec((BLK, D), lambda i: (i, 0)),
            pl.BlockSpec((BLK, 1), lambda i: (i, 0)),
            pl.BlockSpec((BLK, 1), lambda i: (i, 0)),
        ],
        out_shape=[
            jax.ShapeDtypeStruct((N, D), jnp.float32),
            jax.ShapeDtypeStruct((N, 1), jnp.float32),
            jax.ShapeDtypeStruct((N, 1), jnp.float32),
        ],
    )(x, W, av)


def _combine(num_ref, den0_ref, den1_ref, b_ref):
    den = den0_ref[...] + den1_ref[...]
    return (num_ref[0] + num_ref[1]) / (den + 1e-16) + b_ref[...]


def _mid_body(num_ref, den0_ref, den1_ref, b_ref, w_ref, av_ref,
              h_ref, as_ref, ad_ref):
    y = jnp.maximum(_combine(num_ref, den0_ref, den1_ref, b_ref), 0.0)
    h = jnp.dot(y, w_ref[...], preferred_element_type=jnp.float32)
    h_ref[...] = h
    as_ref[...] = jnp.sum(h * av_ref[0:1, :], axis=1, keepdims=True)
    ad_ref[...] = jnp.sum(h * av_ref[1:2, :], axis=1, keepdims=True)


def _mid_call(num, den, b, W, av):
    den0 = den[0, 0, :N].reshape(N, 1)
    den1 = den[1, 0, :N].reshape(N, 1)
    return pl.pallas_call(
        _mid_body,
        grid=(N // BLK,),
        in_specs=[
            pl.BlockSpec((NC, BLK, D), lambda i: (0, i, 0)),
            pl.BlockSpec((BLK, 1), lambda i: (i, 0)),
            pl.BlockSpec((BLK, 1), lambda i: (i, 0)),
            pl.BlockSpec((1, D), lambda i: (0, 0)),
            pl.BlockSpec((D, D), lambda i: (0, 0)),
            pl.BlockSpec((2, D), lambda i: (0, 0)),
        ],
        out_specs=[
            pl.BlockSpec((BLK, D), lambda i: (i, 0)),
            pl.BlockSpec((BLK, 1), lambda i: (i, 0)),
            pl.BlockSpec((BLK, 1), lambda i: (i, 0)),
        ],
        out_shape=[
            jax.ShapeDtypeStruct((N, D), jnp.float32),
            jax.ShapeDtypeStruct((N, 1), jnp.float32),
            jax.ShapeDtypeStruct((N, 1), jnp.float32),
        ],
    )(num, den0, den1, b, W, av)


def _fin_body(num_ref, den0_ref, den1_ref, b_ref, out_ref):
    out_ref[...] = _combine(num_ref, den0_ref, den1_ref, b_ref)


def _fin_call(num, den, b):
    den0 = den[0, 0, :N].reshape(N, 1)
    den1 = den[1, 0, :N].reshape(N, 1)
    return pl.pallas_call(
        _fin_body,
        grid=(N // BLK,),
        in_specs=[
            pl.BlockSpec((NC, BLK, D), lambda i: (0, i, 0)),
            pl.BlockSpec((BLK, 1), lambda i: (i, 0)),
            pl.BlockSpec((BLK, 1), lambda i: (i, 0)),
            pl.BlockSpec((1, D), lambda i: (0, 0)),
        ],
        out_specs=pl.BlockSpec((BLK, D), lambda i: (i, 0)),
        out_shape=jax.ShapeDtypeStruct((N, D), jnp.float32),
    )(num, den0, den1, b)


_MESH = plsc.VectorSubcoreMesh(core_axis_name="c", subcore_axis_name="s",
                               num_cores=NC, num_subcores=NS)


@functools.partial(
    pl.kernel,
    out_type=(
        pltpu.HBM((NC, N, D), jnp.float32),
        pltpu.HBM((NC, 1, NP), jnp.float32),
    ),
    mesh=_MESH,
    compiler_params=pltpu.CompilerParams(needs_layout_passes=False),
    scratch_types=[
        pltpu.VMEM((2, CHUNK), jnp.int32),
        pltpu.VMEM((2, CHUNK), jnp.int32),
        pltpu.VMEM((CHUNK, D), jnp.float32),
        pltpu.VMEM((CHUNK, D), jnp.float32),
        pltpu.VMEM((CHUNK,), jnp.float32),
        pltpu.VMEM((CHUNK,), jnp.float32),
        pltpu.VMEM((CHUNK,), jnp.float32),
        pltpu.VMEM_SHARED((N, D), jnp.float32),
        pltpu.VMEM_SHARED((NP,), jnp.float32),
        pltpu.VMEM_SHARED((NP,), jnp.float32),
        pltpu.VMEM_SHARED((NP,), jnp.float32),
        pltpu.SemaphoreType.DMA,
        pltpu.SemaphoreType.DMA,
        pltpu.SemaphoreType.DMA,
        pltpu.SemaphoreType.DMA,
    ],
)
def _edge_kernel(h_hbm, asrc_hbm, adst_hbm, eidx_hbm,
                 num_hbm, den_hbm,
                 idx0, idx1, rows0, rows1, av_b, bv_b, ex_b,
                 num_sh, asrc_sh, adst_sh, den_sh, gsem0, gsem1, ssem0, ssem1):
    cid = lax.axis_index("c")
    sid = lax.axis_index("s")
    wid = cid * NS + sid

    def _ld(q, _):
        qs = pl.ds(sid * (NP // NS) + q * 128, 128)
        pltpu.sync_copy(asrc_hbm.at[qs], asrc_sh.at[qs])
        pltpu.sync_copy(adst_hbm.at[qs], adst_sh.at[qs])
        return 0
    lax.fori_loop(0, NP // NS // 128, _ld, 0)

    zeros16 = jnp.zeros((L,), jnp.float32)

    def _zrow(i, _):
        for j in range(D // L):
            rows0[i, pl.ds(j * L, L)] = zeros16
        return 0
    lax.fori_loop(0, CHUNK, _zrow, 0)
    base = sid * STRIPE

    def _zsh(i, _):
        pltpu.sync_copy(rows0.at[pl.ds(0, 48)], num_sh.at[pl.ds(base + i * 48, 48)])
        return 0
    lax.fori_loop(0, STRIPE // 48, _zsh, 0)

    @pl.when(sid == NS - 1)
    def _():
        pltpu.sync_copy(rows0.at[pl.ds(0, 16)],
                        num_sh.at[pl.ds(NS * STRIPE, N - NS * STRIPE)])

    def _zden(i, _):
        pltpu.sync_copy(rows0.at[0], den_sh.at[pl.ds(sid * 640 + i * 128, 128)])
        return 0
    lax.fori_loop(0, 5, _zden, 0)
    plsc.subcore_barrier()

    idxs = (idx0, idx1)
    rows = (rows0, rows1)
    gsems = (gsem0, gsem1)
    ssems = (ssem0, ssem1)

    pltpu.sync_copy(eidx_hbm.at[wid, 0], idx0)
    pltpu.async_copy(h_hbm.at[idx0.at[0]], rows0, gsem0)

    def _chunk(i, _):
        for b in range(2):
            c = 2 * i + b
            nb = (b + 1) % 2
            idx_b = idxs[b]

            @pl.when(c < NCHUNK)
            def _():
                s_row = idx_b.at[0]
                d_row = idx_b.at[1]

                pltpu.sync_copy(asrc_sh.at[s_row], av_b)
                pltpu.sync_copy(adst_sh.at[d_row], bv_b)
                for k in range(CHUNK // L):
                    e = av_b[pl.ds(k * L, L)] + bv_b[pl.ds(k * L, L)]
                    e = jnp.where(e >= 0.0, e, 0.2 * e)
                    ex_b[pl.ds(k * L, L)] = jnp.exp(e)
                pltpu.sync_copy(ex_b, den_sh.at[d_row], add=True)

                @pl.when(c + 1 < NCHUNK)
                def _():
                    @pl.when(c >= 1)
                    def _():
                        pltpu.make_async_copy(
                            rows[nb], num_sh.at[idxs[nb].at[1]],
                            ssems[nb]).wait()
                    pltpu.sync_copy(eidx_hbm.at[wid, c + 1], idxs[nb])
                    pltpu.async_copy(h_hbm.at[idxs[nb].at[0]],
                                     rows[nb], gsems[nb])

                pltpu.make_async_copy(h_hbm.at[s_row], rows[b],
                                      gsems[b]).wait()

                def _scale(e_i, _):
                    exs = plsc.load_gather(ex_b, [jnp.full((L,), e_i, jnp.int32)])
                    for j in range(D // L):
                        rows[b][e_i, pl.ds(j * L, L)] = (
                            rows[b][e_i, pl.ds(j * L, L)] * exs)
                    return 0
                lax.fori_loop(0, CHUNK, _scale, 0)

                pltpu.async_copy(rows[b], num_sh.at[d_row], ssems[b],
                                 add=True)
        return 0
    lax.fori_loop(0, (NCHUNK + 1) // 2, _chunk, 0)

    pltpu.make_async_copy(rows1, num_sh.at[idx1.at[1]], ssem1).wait()
    pltpu.make_async_copy(rows0, num_sh.at[idx0.at[1]], ssem0).wait()

    plsc.subcore_barrier()

    def _wout(q, _):
        qs = pl.ds(base + q * 48, 48)
        pltpu.sync_copy(num_sh.at[qs], num_hbm.at[cid, qs])
        return 0
    lax.fori_loop(0, STRIPE // 48, _wout, 0)

    @pl.when(sid == NS - 1)
    def _():
        qs = pl.ds(NS * STRIPE, N - NS * STRIPE)
        pltpu.sync_copy(num_sh.at[qs], num_hbm.at[cid, qs])

    def _wden(q, _):
        qs = pl.ds(sid * 640 + q * 128, 128)
        pltpu.sync_copy(den_sh.at[qs], den_hbm.at[cid, 0, qs])
        return 0
    lax.fori_loop(0, 5, _wden, 0)


def kernel(x, edge_index, W1, a1_src, a1_dst, b1, W2, a2_src, a2_dst, b2):
    eidx = jnp.stack([edge_index[0].reshape(NW, NCHUNK, CHUNK),
                      edge_index[1].reshape(NW, NCHUNK, CHUNK)], axis=2)
    pad = (0, NP - N)

    h1, as1, ad1 = _pre_call(x, W1, jnp.stack([a1_src, a1_dst]))
    num1, den1 = _edge_kernel(h1, jnp.pad(as1.reshape(N), pad),
                              jnp.pad(ad1.reshape(N), pad), eidx)
    h2, as2, ad2 = _mid_call(num1, den1, b1.reshape(1, D), W2,
                             jnp.stack([a2_src, a2_dst]))
    num2, den2 = _edge_kernel(h2, jnp.pad(as2.reshape(N), pad),
                              jnp.pad(ad2.reshape(N), pad), eidx)
    return _fin_call(num2, den2, b2.reshape(1, D))


# speedup vs baseline: 1.1640x; 1.1640x over previous
import functools

import jax
import jax.numpy as jnp
from jax import lax
from jax.experimental import pallas as pl
from jax.experimental.pallas import tpu as pltpu
from jax.experimental.pallas import tpu_sc as plsc

N = 10000
NP = 10240
E = 320000
D = 128
NC = 2
NS = 16
NW = NC * NS
EPT = E // NW
CHUNK = 80
NCHUNK = EPT // CHUNK
STRIPE = 624
L = 16


BLK = 2000


def _pre_body(x_ref, w_ref, av_ref, h_ref, as_ref, ad_ref):
    h = jnp.dot(x_ref[...], w_ref[...], preferred_element_type=jnp.float32)
    h_ref[...] = h
    as_ref[...] = jnp.sum(h * av_ref[0:1, :], axis=1, keepdims=True)
    ad_ref[...] = jnp.sum(h * av_ref[1:2, :], axis=1, keepdims=True)


def _pre_call(x, W, av):
    return pl.pallas_call(
        _pre_body,
        grid=(N // BLK,),
        in_specs=[
            pl.BlockSpec((BLK, D), lambda i: (i, 0)),
            pl.BlockSpec((D, D), lambda i: (0, 0)),
            pl.BlockSpec((2, D), lambda i: (0, 0)),
        ],
        out_specs=[
            pl.BlockSpec((BLK, D), lambda i: (i, 0)),
            pl.BlockSpec((BLK, 1), lambda i: (i, 0)),
            pl.BlockSpec((BLK, 1), lambda i: (i, 0)),
        ],
        out_shape=[
            jax.ShapeDtypeStruct((N, D), jnp.float32),
            jax.ShapeDtypeStruct((N, 1), jnp.float32),
            jax.ShapeDtypeStruct((N, 1), jnp.float32),
        ],
    )(x, W, av)


def _combine(num_ref, den0_ref, den1_ref, b_ref):
    den = den0_ref[...] + den1_ref[...]
    return (num_ref[0] + num_ref[1]) / (den + 1e-16) + b_ref[...]


def _mid_body(num_ref, den0_ref, den1_ref, b_ref, w_ref, av_ref,
              h_ref, as_ref, ad_ref):
    y = jnp.maximum(_combine(num_ref, den0_ref, den1_ref, b_ref), 0.0)
    h = jnp.dot(y, w_ref[...], preferred_element_type=jnp.float32)
    h_ref[...] = h
    as_ref[...] = jnp.sum(h * av_ref[0:1, :], axis=1, keepdims=True)
    ad_ref[...] = jnp.sum(h * av_ref[1:2, :], axis=1, keepdims=True)


def _mid_call(num, den, b, W, av):
    den0 = den[0, 0, :N].reshape(N, 1)
    den1 = den[1, 0, :N].reshape(N, 1)
    return pl.pallas_call(
        _mid_body,
        grid=(N // BLK,),
        in_specs=[
            pl.BlockSpec((NC, BLK, D), lambda i: (0, i, 0)),
            pl.BlockSpec((BLK, 1), lambda i: (i, 0)),
            pl.BlockSpec((BLK, 1), lambda i: (i, 0)),
            pl.BlockSpec((1, D), lambda i: (0, 0)),
            pl.BlockSpec((D, D), lambda i: (0, 0)),
            pl.BlockSpec((2, D), lambda i: (0, 0)),
        ],
        out_specs=[
            pl.BlockSpec((BLK, D), lambda i: (i, 0)),
            pl.BlockSpec((BLK, 1), lambda i: (i, 0)),
            pl.BlockSpec((BLK, 1), lambda i: (i, 0)),
        ],
        out_shape=[
            jax.ShapeDtypeStruct((N, D), jnp.float32),
            jax.ShapeDtypeStruct((N, 1), jnp.float32),
            jax.ShapeDtypeStruct((N, 1), jnp.float32),
        ],
    )(num, den0, den1, b, W, av)


def _fin_body(num_ref, den0_ref, den1_ref, b_ref, out_ref):
    out_ref[...] = _combine(num_ref, den0_ref, den1_ref, b_ref)


def _fin_call(num, den, b):
    den0 = den[0, 0, :N].reshape(N, 1)
    den1 = den[1, 0, :N].reshape(N, 1)
    return pl.pallas_call(
        _fin_body,
        grid=(N // BLK,),
        in_specs=[
            pl.BlockSpec((NC, BLK, D), lambda i: (0, i, 0)),
            pl.BlockSpec((BLK, 1), lambda i: (i, 0)),
            pl.BlockSpec((BLK, 1), lambda i: (i, 0)),
            pl.BlockSpec((1, D), lambda i: (0, 0)),
        ],
        out_specs=pl.BlockSpec((BLK, D), lambda i: (i, 0)),
        out_shape=jax.ShapeDtypeStruct((N, D), jnp.float32),
    )(num, den0, den1, b)


_MESH = plsc.VectorSubcoreMesh(core_axis_name="c", subcore_axis_name="s",
                               num_cores=NC, num_subcores=NS)


@functools.partial(
    pl.kernel,
    out_type=(
        pltpu.HBM((NC, N, D), jnp.float32),
        pltpu.HBM((NC, 1, NP), jnp.float32),
    ),
    mesh=_MESH,
    compiler_params=pltpu.CompilerParams(needs_layout_passes=False),
    scratch_types=[
        pltpu.VMEM((8, 2, CHUNK), jnp.int32),
        pltpu.VMEM((8, 2, CHUNK), jnp.int32),
        pltpu.VMEM((CHUNK, D), jnp.float32),
        pltpu.VMEM((CHUNK, D), jnp.float32),
        pltpu.VMEM((CHUNK,), jnp.float32),
        pltpu.VMEM((CHUNK,), jnp.float32),
        pltpu.VMEM((CHUNK,), jnp.float32),
        pltpu.VMEM_SHARED((N, D), jnp.float32),
        pltpu.VMEM_SHARED((NP,), jnp.float32),
        pltpu.VMEM_SHARED((NP,), jnp.float32),
        pltpu.VMEM_SHARED((NP,), jnp.float32),
        pltpu.SemaphoreType.DMA,
        pltpu.SemaphoreType.DMA,
        pltpu.SemaphoreType.DMA,
        pltpu.SemaphoreType.DMA,
    ],
)
def _edge_kernel(h_hbm, asrc_hbm, adst_hbm, eidx_hbm,
                 num_hbm, den_hbm,
                 sb0, sb1, rows0, rows1, av_b, bv_b, ex_b,
                 num_sh, asrc_sh, adst_sh, den_sh, gsem0, gsem1, ssem0, ssem1):
    cid = lax.axis_index("c")
    sid = lax.axis_index("s")
    wid = cid * NS + sid

    def _ld(q, _):
        qs = pl.ds(sid * (NP // NS) + q * 128, 128)
        pltpu.sync_copy(asrc_hbm.at[qs], asrc_sh.at[qs])
        pltpu.sync_copy(adst_hbm.at[qs], adst_sh.at[qs])
        return 0
    lax.fori_loop(0, NP // NS // 128, _ld, 0)

    zeros16 = jnp.zeros((L,), jnp.float32)

    def _zrow(i, _):
        for j in range(D // L):
            rows0[i, pl.ds(j * L, L)] = zeros16
        return 0
    lax.fori_loop(0, CHUNK, _zrow, 0)
    base = sid * STRIPE

    def _zsh(i, _):
        pltpu.sync_copy(rows0.at[pl.ds(0, 48)], num_sh.at[pl.ds(base + i * 48, 48)])
        return 0
    lax.fori_loop(0, STRIPE // 48, _zsh, 0)

    @pl.when(sid == NS - 1)
    def _():
        pltpu.sync_copy(rows0.at[pl.ds(0, 16)],
                        num_sh.at[pl.ds(NS * STRIPE, N - NS * STRIPE)])

    def _zden(i, _):
        pltpu.sync_copy(rows0.at[0], den_sh.at[pl.ds(sid * 640 + i * 128, 128)])
        return 0
    lax.fori_loop(0, 5, _zden, 0)
    plsc.subcore_barrier()

    sbufs = (sb0, sb1)
    rows = (rows0, rows1)
    gsems = (gsem0, gsem1)
    ssems = (ssem0, ssem1)

    def _super(si, _):
        sbuf = sbufs[0]
        sprev = sbufs[1]
        for half in range(2):
            s2 = 2 * si + half
            if half == 1:
                sbuf, sprev = sprev, sbuf

            @pl.when(s2 * 8 < NCHUNK)
            def _():
                @pl.when(s2 >= 1)
                def _():
                    pltpu.make_async_copy(
                        rows[0], num_sh.at[sbuf.at[0, 1]], ssems[0]).wait()
                pltpu.sync_copy(eidx_hbm.at[wid, s2], sbuf)
                pltpu.async_copy(h_hbm.at[sbuf.at[0, 0]], rows[0], gsems[0])

                for b in range(8):
                    c = s2 * 8 + b
                    p = b % 2
                    np_ = (b + 1) % 2

                    @pl.when(c < NCHUNK)
                    def _():
                        s_row = sbuf.at[b, 0]
                        d_row = sbuf.at[b, 1]

                        pltpu.sync_copy(asrc_sh.at[s_row], av_b)
                        pltpu.sync_copy(adst_sh.at[d_row], bv_b)
                        for k in range(CHUNK // L):
                            e = av_b[pl.ds(k * L, L)] + bv_b[pl.ds(k * L, L)]
                            e = jnp.where(e >= 0.0, e, 0.2 * e)
                            ex_b[pl.ds(k * L, L)] = jnp.exp(e)
                        pltpu.sync_copy(ex_b, den_sh.at[d_row], add=True)

                        if b < 7:
                            @pl.when(c + 1 < NCHUNK)
                            def _():
                                @pl.when(c >= 1)
                                def _():
                                    pltpu.make_async_copy(
                                        rows[np_],
                                        num_sh.at[sbuf.at[b + 1, 1]],
                                        ssems[np_]).wait()
                                pltpu.async_copy(h_hbm.at[sbuf.at[b + 1, 0]],
                                                 rows[np_], gsems[np_])

                        pltpu.make_async_copy(h_hbm.at[s_row], rows[p],
                                              gsems[p]).wait()

                        def _scale(e_i, _):
                            exs = plsc.load_gather(
                                ex_b, [jnp.full((L,), e_i, jnp.int32)])
                            for j in range(D // L):
                                rows[p][e_i, pl.ds(j * L, L)] = (
                                    rows[p][e_i, pl.ds(j * L, L)] * exs)
                            return 0
                        lax.fori_loop(0, CHUNK, _scale, 0)

                        pltpu.async_copy(rows[p], num_sh.at[d_row],
                                         ssems[p], add=True)
        return 0
    lax.fori_loop(0, 8, _super, 0)

    pltpu.make_async_copy(rows1, num_sh.at[sb0.at[0, 1]], ssem1).wait()
    pltpu.make_async_copy(rows0, num_sh.at[sb0.at[0, 1]], ssem0).wait()

    plsc.subcore_barrier()

    def _wout(q, _):
        qs = pl.ds(base + q * 48, 48)
        pltpu.sync_copy(num_sh.at[qs], num_hbm.at[cid, qs])
        return 0
    lax.fori_loop(0, STRIPE // 48, _wout, 0)

    @pl.when(sid == NS - 1)
    def _():
        qs = pl.ds(NS * STRIPE, N - NS * STRIPE)
        pltpu.sync_copy(num_sh.at[qs], num_hbm.at[cid, qs])

    def _wden(q, _):
        qs = pl.ds(sid * 640 + q * 128, 128)
        pltpu.sync_copy(den_sh.at[qs], den_hbm.at[cid, 0, qs])
        return 0
    lax.fori_loop(0, 5, _wden, 0)


def kernel(x, edge_index, W1, a1_src, a1_dst, b1, W2, a2_src, a2_dst, b2):
    eidx = jnp.stack([edge_index[0].reshape(NW, NCHUNK, CHUNK),
                      edge_index[1].reshape(NW, NCHUNK, CHUNK)], axis=2)
    eidx = jnp.pad(eidx, ((0, 0), (0, 128 - NCHUNK), (0, 0), (0, 0)))
    eidx = eidx.reshape(NW, 16, 8, 2, CHUNK)
    pad = (0, NP - N)

    h1, as1, ad1 = _pre_call(x, W1, jnp.stack([a1_src, a1_dst]))
    num1, den1 = _edge_kernel(h1, jnp.pad(as1.reshape(N), pad),
                              jnp.pad(ad1.reshape(N), pad), eidx)
    h2, as2, ad2 = _mid_call(num1, den1, b1.reshape(1, D), W2,
                             jnp.stack([a2_src, a2_dst]))
    num2, den2 = _edge_kernel(h2, jnp.pad(as2.reshape(N), pad),
                              jnp.pad(ad2.reshape(N), pad), eidx)
    return _fin_call(num2, den2, b2.reshape(1, D))


# speedup vs baseline: 1.1896x; 1.0219x over previous
import functools

import jax
import jax.numpy as jnp
from jax import lax
from jax.experimental import pallas as pl
from jax.experimental.pallas import tpu as pltpu
from jax.experimental.pallas import tpu_sc as plsc

N = 10000
NP = 10240
E = 320000
D = 128
NC = 2
NS = 16
NW = NC * NS
EPT = E // NW
CHUNK = 80
NCHUNK = EPT // CHUNK
STRIPE = 624
L = 16


BLK = 2000


def _pre_body(x_ref, w_ref, av_ref, h_ref, as_ref, ad_ref):
    h = jnp.dot(x_ref[...], w_ref[...], preferred_element_type=jnp.float32)
    h_ref[...] = h
    as_ref[...] = jnp.sum(h * av_ref[0:1, :], axis=1, keepdims=True)
    ad_ref[...] = jnp.sum(h * av_ref[1:2, :], axis=1, keepdims=True)


def _pre_call(x, W, av):
    return pl.pallas_call(
        _pre_body,
        grid=(N // BLK,),
        in_specs=[
            pl.BlockSpec((BLK, D), lambda i: (i, 0)),
            pl.BlockSpec((D, D), lambda i: (0, 0)),
            pl.BlockSpec((2, D), lambda i: (0, 0)),
        ],
        out_specs=[
            pl.BlockSpec((BLK, D), lambda i: (i, 0)),
            pl.BlockSpec((BLK, 1), lambda i: (i, 0)),
            pl.BlockSpec((BLK, 1), lambda i: (i, 0)),
        ],
        out_shape=[
            jax.ShapeDtypeStruct((N, D), jnp.float32),
            jax.ShapeDtypeStruct((N, 1), jnp.float32),
            jax.ShapeDtypeStruct((N, 1), jnp.float32),
        ],
    )(x, W, av)


def _combine(num_ref, den0_ref, den1_ref, b_ref):
    den = den0_ref[...] + den1_ref[...]
    return (num_ref[0] + num_ref[1]) / (den + 1e-16) + b_ref[...]


def _mid_body(num_ref, den0_ref, den1_ref, b_ref, w_ref, av_ref,
              h_ref, as_ref, ad_ref):
    y = jnp.maximum(_combine(num_ref, den0_ref, den1_ref, b_ref), 0.0)
    h = jnp.dot(y, w_ref[...], preferred_element_type=jnp.float32)
    h_ref[...] = h
    as_ref[...] = jnp.sum(h * av_ref[0:1, :], axis=1, keepdims=True)
    ad_ref[...] = jnp.sum(h * av_ref[1:2, :], axis=1, keepdims=True)


def _mid_call(num, den, b, W, av):
    den0 = den[0, 0, :N].reshape(N, 1)
    den1 = den[1, 0, :N].reshape(N, 1)
    return pl.pallas_call(
        _mid_body,
        grid=(N // BLK,),
        in_specs=[
            pl.BlockSpec((NC, BLK, D), lambda i: (0, i, 0)),
            pl.BlockSpec((BLK, 1), lambda i: (i, 0)),
            pl.BlockSpec((BLK, 1), lambda i: (i, 0)),
            pl.BlockSpec((1, D), lambda i: (0, 0)),
            pl.BlockSpec((D, D), lambda i: (0, 0)),
            pl.BlockSpec((2, D), lambda i: (0, 0)),
        ],
        out_specs=[
            pl.BlockSpec((BLK, D), lambda i: (i, 0)),
            pl.BlockSpec((BLK, 1), lambda i: (i, 0)),
            pl.BlockSpec((BLK, 1), lambda i: (i, 0)),
        ],
        out_shape=[
            jax.ShapeDtypeStruct((N, D), jnp.float32),
            jax.ShapeDtypeStruct((N, 1), jnp.float32),
            jax.ShapeDtypeStruct((N, 1), jnp.float32),
        ],
    )(num, den0, den1, b, W, av)


def _fin_body(num_ref, den0_ref, den1_ref, b_ref, out_ref):
    out_ref[...] = _combine(num_ref, den0_ref, den1_ref, b_ref)


def _fin_call(num, den, b):
    den0 = den[0, 0, :N].reshape(N, 1)
    den1 = den[1, 0, :N].reshape(N, 1)
    return pl.pallas_call(
        _fin_body,
        grid=(N // BLK,),
        in_specs=[
            pl.BlockSpec((NC, BLK, D), lambda i: (0, i, 0)),
            pl.BlockSpec((BLK, 1), lambda i: (i, 0)),
            pl.BlockSpec((BLK, 1), lambda i: (i, 0)),
            pl.BlockSpec((1, D), lambda i: (0, 0)),
        ],
        out_specs=pl.BlockSpec((BLK, D), lambda i: (i, 0)),
        out_shape=jax.ShapeDtypeStruct((N, D), jnp.float32),
    )(num, den0, den1, b)


_MESH = plsc.VectorSubcoreMesh(core_axis_name="c", subcore_axis_name="s",
                               num_cores=NC, num_subcores=NS)


@functools.partial(
    pl.kernel,
    out_type=(
        pltpu.HBM((NC, N, D), jnp.float32),
        pltpu.HBM((NC, 1, NP), jnp.float32),
    ),
    mesh=_MESH,
    compiler_params=pltpu.CompilerParams(needs_layout_passes=False),
    scratch_types=[
        pltpu.VMEM((8, 2, CHUNK), jnp.int32),
        pltpu.VMEM((8, 2, CHUNK), jnp.int32),
        pltpu.VMEM((CHUNK, D), jnp.float32),
        pltpu.VMEM((CHUNK, D), jnp.float32),
        pltpu.VMEM((CHUNK,), jnp.float32),
        pltpu.VMEM((CHUNK,), jnp.float32),
        pltpu.VMEM((CHUNK,), jnp.float32),
        pltpu.VMEM((CHUNK,), jnp.float32),
        pltpu.VMEM((CHUNK,), jnp.float32),
        pltpu.VMEM((CHUNK,), jnp.float32),
        pltpu.VMEM_SHARED((N, D), jnp.float32),
        pltpu.VMEM_SHARED((NP,), jnp.float32),
        pltpu.VMEM_SHARED((NP,), jnp.float32),
        pltpu.VMEM_SHARED((NP,), jnp.float32),
        pltpu.SemaphoreType.DMA,
        pltpu.SemaphoreType.DMA,
        pltpu.SemaphoreType.DMA,
        pltpu.SemaphoreType.DMA,
        pltpu.SemaphoreType.DMA,
        pltpu.SemaphoreType.DMA,
        pltpu.SemaphoreType.DMA,
        pltpu.SemaphoreType.DMA,
    ],
)
def _edge_kernel(h_hbm, asrc_hbm, adst_hbm, eidx_hbm,
                 num_hbm, den_hbm,
                 sb0, sb1, rows0, rows1, av0, av1, bv0, bv1, ex0, ex1,
                 num_sh, asrc_sh, adst_sh, den_sh,
                 gsem0, gsem1, ssem0, ssem1, asem0, asem1, dsem0, dsem1):
    cid = lax.axis_index("c")
    sid = lax.axis_index("s")
    wid = cid * NS + sid

    def _ld(q, _):
        qs = pl.ds(sid * (NP // NS) + q * 128, 128)
        pltpu.sync_copy(asrc_hbm.at[qs], asrc_sh.at[qs])
        pltpu.sync_copy(adst_hbm.at[qs], adst_sh.at[qs])
        return 0
    lax.fori_loop(0, NP // NS // 128, _ld, 0)

    zeros16 = jnp.zeros((L,), jnp.float32)

    def _zrow(i, _):
        for j in range(D // L):
            rows0[i, pl.ds(j * L, L)] = zeros16
        return 0
    lax.fori_loop(0, CHUNK, _zrow, 0)
    base = sid * STRIPE

    def _zsh(i, _):
        pltpu.sync_copy(rows0.at[pl.ds(0, 48)], num_sh.at[pl.ds(base + i * 48, 48)])
        return 0
    lax.fori_loop(0, STRIPE // 48, _zsh, 0)

    @pl.when(sid == NS - 1)
    def _():
        pltpu.sync_copy(rows0.at[pl.ds(0, 16)],
                        num_sh.at[pl.ds(NS * STRIPE, N - NS * STRIPE)])

    def _zden(i, _):
        pltpu.sync_copy(rows0.at[0], den_sh.at[pl.ds(sid * 640 + i * 128, 128)])
        return 0
    lax.fori_loop(0, 5, _zden, 0)
    plsc.subcore_barrier()

    sbufs = (sb0, sb1)
    rows = (rows0, rows1)
    avs = (av0, av1)
    bvs = (bv0, bv1)
    exs_ = (ex0, ex1)
    gsems = (gsem0, gsem1)
    ssems = (ssem0, ssem1)
    asems = (asem0, asem1)
    dsems = (dsem0, dsem1)

    def _super(si, _):
        sbuf = sbufs[0]
        sprev = sbufs[1]
        for half in range(2):
            s2 = 2 * si + half
            if half == 1:
                sbuf, sprev = sprev, sbuf

            @pl.when(s2 * 8 < NCHUNK)
            def _():
                @pl.when(s2 >= 1)
                def _():
                    pltpu.make_async_copy(
                        rows[0], num_sh.at[sbuf.at[0, 1]], ssems[0]).wait()
                pltpu.sync_copy(eidx_hbm.at[wid, s2], sbuf)
                pltpu.async_copy(h_hbm.at[sbuf.at[0, 0]], rows[0], gsems[0])
                pltpu.async_copy(asrc_sh.at[sbuf.at[0, 0]], avs[0], asems[0])
                pltpu.async_copy(adst_sh.at[sbuf.at[0, 1]], bvs[0], asems[0])

                for b in range(8):
                    c = s2 * 8 + b
                    p = b % 2
                    np_ = (b + 1) % 2

                    @pl.when(c < NCHUNK)
                    def _():
                        s_row = sbuf.at[b, 0]
                        d_row = sbuf.at[b, 1]
                        av_b, bv_b, ex_b = avs[p], bvs[p], exs_[p]

                        pltpu.make_async_copy(asrc_sh.at[s_row], av_b,
                                              asems[p]).wait()
                        pltpu.make_async_copy(adst_sh.at[d_row], bv_b,
                                              asems[p]).wait()

                        @pl.when(c >= 2)
                        def _():
                            pltpu.make_async_copy(
                                ex_b, den_sh.at[d_row], dsems[p]).wait()
                        for k in range(CHUNK // L):
                            e = av_b[pl.ds(k * L, L)] + bv_b[pl.ds(k * L, L)]
                            e = jnp.where(e >= 0.0, e, 0.2 * e)
                            ex_b[pl.ds(k * L, L)] = jnp.exp(e)
                        pltpu.async_copy(ex_b, den_sh.at[d_row], dsems[p],
                                         add=True)

                        if b < 7:
                            @pl.when(c + 1 < NCHUNK)
                            def _():
                                @pl.when(c >= 1)
                                def _():
                                    pltpu.make_async_copy(
                                        rows[np_],
                                        num_sh.at[sbuf.at[b + 1, 1]],
                                        ssems[np_]).wait()
                                pltpu.async_copy(h_hbm.at[sbuf.at[b + 1, 0]],
                                                 rows[np_], gsems[np_])
                                pltpu.async_copy(asrc_sh.at[sbuf.at[b + 1, 0]],
                                                 avs[np_], asems[np_])
                                pltpu.async_copy(adst_sh.at[sbuf.at[b + 1, 1]],
                                                 bvs[np_], asems[np_])

                        pltpu.make_async_copy(h_hbm.at[s_row], rows[p],
                                              gsems[p]).wait()

                        def _scale(e_i, _):
                            for u in range(2):
                                ei = e_i * 2 + u
                                exv = plsc.load_gather(
                                    ex_b, [jnp.full((L,), ei, jnp.int32)])
                                for j in range(D // L):
                                    rows[p][ei, pl.ds(j * L, L)] = (
                                        rows[p][ei, pl.ds(j * L, L)] * exv)
                            return 0
                        lax.fori_loop(0, CHUNK // 2, _scale, 0)

                        pltpu.async_copy(rows[p], num_sh.at[d_row],
                                         ssems[p], add=True)
        return 0
    lax.fori_loop(0, 8, _super, 0)

    pltpu.make_async_copy(rows1, num_sh.at[sb0.at[0, 1]], ssem1).wait()
    pltpu.make_async_copy(rows0, num_sh.at[sb0.at[0, 1]], ssem0).wait()
    pltpu.make_async_copy(ex1, den_sh.at[sb0.at[0, 1]], dsem1).wait()
    pltpu.make_async_copy(ex0, den_sh.at[sb0.at[0, 1]], dsem0).wait()

    plsc.subcore_barrier()

    def _wout(q, _):
        qs = pl.ds(base + q * 48, 48)
        pltpu.sync_copy(num_sh.at[qs], num_hbm.at[cid, qs])
        return 0
    lax.fori_loop(0, STRIPE // 48, _wout, 0)

    @pl.when(sid == NS - 1)
    def _():
        qs = pl.ds(NS * STRIPE, N - NS * STRIPE)
        pltpu.sync_copy(num_sh.at[qs], num_hbm.at[cid, qs])

    def _wden(q, _):
        qs = pl.ds(sid * 640 + q * 128, 128)
        pltpu.sync_copy(den_sh.at[qs], den_hbm.at[cid, 0, qs])
        return 0
    lax.fori_loop(0, 5, _wden, 0)


def kernel(x, edge_index, W1, a1_src, a1_dst, b1, W2, a2_src, a2_dst, b2):
    eidx = jnp.stack([edge_index[0].reshape(NW, NCHUNK, CHUNK),
                      edge_index[1].reshape(NW, NCHUNK, CHUNK)], axis=2)
    eidx = jnp.pad(eidx, ((0, 0), (0, 128 - NCHUNK), (0, 0), (0, 0)))
    eidx = eidx.reshape(NW, 16, 8, 2, CHUNK)
    pad = (0, NP - N)

    h1, as1, ad1 = _pre_call(x, W1, jnp.stack([a1_src, a1_dst]))
    num1, den1 = _edge_kernel(h1, jnp.pad(as1.reshape(N), pad),
                              jnp.pad(ad1.reshape(N), pad), eidx)
    h2, as2, ad2 = _mid_call(num1, den1, b1.reshape(1, D), W2,
                             jnp.stack([a2_src, a2_dst]))
    num2, den2 = _edge_kernel(h2, jnp.pad(as2.reshape(N), pad),
                              jnp.pad(ad2.reshape(N), pad), eidx)
    return _fin_call(num2, den2, b2.reshape(1, D))
